# baseline (device time: 76612 ns/iter reference)
import jax
import jax.numpy as jnp
from jax import lax
from jax.experimental import pallas as pl
from jax.experimental.pallas import tpu as pltpu

N_DEV = 4
E_LOC = 4
E = N_DEV * E_LOC
N_TOK = 1024
HALF = N_TOK // 2
D = 512
H = 1024

KX, KG, KRS = 0, 1, 2


def kernel(x, router_W, route_idx, expert_W):
    ew2 = expert_W.astype(jnp.bfloat16).reshape(E_LOC * D, H)
    x_bf16 = x.astype(jnp.bfloat16)
    xa = x_bf16[:HALF]
    xb = x_bf16[HALF:]

    def body(x_ref, rw_ref, idx_ref, ew_ref, xa_ref, xb_ref, out_ref,
             agx_cw, agx_ccw, agg_cw, agg_ccw, gsrc_cw, gsrc_ccw,
             rs_start_cw, rs_start_ccw, rs_recv_cw, rs_recv_ccw,
             pacc_cw, pacc_ccw, send_cw, recv_cw, send_ccw, recv_ccw,
             rssend_cw, rsrecv_cw, rssend_ccw, rsrecv_ccw,
             xgc_cw, xgc_ccw, xgl):
        my = lax.axis_index("i")
        left = lax.rem(my - 1 + N_DEV, N_DEV)
        right = lax.rem(my + 1, N_DEV)

        barrier_sem = pltpu.get_barrier_semaphore()
        for nbr in (left, right):
            pl.semaphore_signal(
                barrier_sem, inc=1,
                device_id=(nbr,), device_id_type=pl.DeviceIdType.MESH,
            )
        pl.semaphore_wait(barrier_sem, 2)

        def make_copy(kind, h, ccw, src, dst):
            if ccw:
                tgt = left if h < 2 else right
            else:
                tgt = right if h < 2 else left
            return pltpu.make_async_remote_copy(
                src_ref=src,
                dst_ref=dst,
                send_sem=(send_ccw if ccw else send_cw).at[kind, h],
                recv_sem=(recv_ccw if ccw else recv_cw).at[kind, h],
                device_id=(tgt,),
                device_id_type=pl.DeviceIdType.MESH,
            )

        rdmas = {}

        def start(kind, h, ccw, src, dst):
            rd = make_copy(kind, h, ccw, src, dst)
            rdmas[(kind, h, ccw)] = rd
            rd.start()

        HH = H // 2

        def start_rs(h, q, ccw, src, dst):
            rd = pltpu.make_async_remote_copy(
                src_ref=src,
                dst_ref=dst,
                send_sem=(rssend_ccw if ccw else rssend_cw).at[h, q],
                recv_sem=(rsrecv_ccw if ccw else rsrecv_cw).at[h, q],
                device_id=(left if ccw else right,),
                device_id_type=pl.DeviceIdType.MESH,
            )
            rdmas[("rs", h, q, ccw)] = rd
            rd.start()

        start(KX, 0, False, xa_ref, agx_cw.at[0])
        start(KX, 0, True, xb_ref, agx_ccw.at[0])

        sc_gates = jax.named_scope("gates")
        sc_gates.__enter__()
        xf = x_ref[:, :]
        scores = jnp.dot(xf, rw_ref[:, :], preferred_element_type=jnp.float32)
        m = jnp.max(scores, axis=1, keepdims=True)
        p = jnp.exp(scores - m)
        p = p / jnp.sum(p, axis=1, keepdims=True)
        e0 = idx_ref[:, 0:1]
        e1 = idx_ref[:, 1:2]
        lanes = lax.broadcasted_iota(jnp.int32, (N_TOK, E), 1)
        m0 = lanes == e0
        m1 = lanes == e1
        g0 = jnp.sum(jnp.where(m0, p, 0.0), axis=1, keepdims=True)
        g1 = jnp.sum(jnp.where(m1, p, 0.0), axis=1, keepdims=True)
        gs = g0 + g1
        gate = jnp.where(m0, g0 / gs, 0.0) + jnp.where(m1, g1 / gs, 0.0)

        gsrc_cw[:, :] = gate[:HALF].astype(jnp.bfloat16)
        gsrc_ccw[:, :] = gate[HALF:].astype(jnp.bfloat16)
        start(KG, 0, False, gsrc_cw, agg_cw.at[0])
        start(KG, 0, True, gsrc_ccw, agg_ccw.at[0])
        sc_gates.__exit__(None, None, None)

        def gate_block(gr):
            rows = lax.broadcasted_iota(jnp.int32, (E, E_LOC), 0)
            cols = lax.broadcasted_iota(jnp.int32, (E, E_LOC), 1)
            sel = (rows == my * E_LOC + cols).astype(jnp.float32)
            return jnp.dot(gr, sel, preferred_element_type=jnp.float32)

        def compute_pacc(ccw, h):
            agx = agx_ccw if ccw else agx_cw
            agg = agg_ccw if ccw else agg_cw
            pacc = pacc_ccw if ccw else pacc_cw
            xgc = xgc_ccw if ccw else xgc_cw
            xr = agx[h]
            gblk = gate_block(agg[h].astype(jnp.float32))
            for j in range(E_LOC):
                xgc[:, j * D:(j + 1) * D] = (
                    xr * gblk[:, j:j + 1]
                ).astype(jnp.bfloat16)
            pacc[:, :] = jnp.dot(xgc[:, :], ew_ref[:, :],
                                 preferred_element_type=jnp.float32)

        with jax.named_scope("local_block"):
            gblk = gate_block(gate)
            for j in range(E_LOC):
                xgl[:, j * D:(j + 1) * D] = (
                    xf * gblk[:, j:j + 1]
                ).astype(jnp.bfloat16)
            out_ref[:, :] = jnp.dot(xgl[:, :], ew_ref[:, :],
                                    preferred_element_type=jnp.float32)

        for h in range(N_DEV - 1):
            for ccw in (False, True):
                agx = agx_ccw if ccw else agx_cw
                agg = agg_ccw if ccw else agg_cw
                with jax.named_scope(f"agwait#h={h}#d={int(ccw)}"):
                    rdmas[(KX, h, ccw)].wait_recv()
                    rdmas[(KG, h, ccw)].wait_recv()
                if h == 0:
                    start(KX, 1, ccw, agx.at[0], agx.at[1])
                    start(KG, 1, ccw, agg.at[0], agg.at[1])
            if h == 0:
                start(KX, 2, False, xa_ref, agx_cw.at[2])
                start(KX, 2, True, xb_ref, agx_ccw.at[2])
                start(KG, 2, False, gsrc_cw, agg_cw.at[2])
                start(KG, 2, True, gsrc_ccw, agg_ccw.at[2])
            for ccw in (False, True):
                pacc = pacc_ccw if ccw else pacc_cw
                rs_start = rs_start_ccw if ccw else rs_start_cw
                rs_recv = rs_recv_ccw if ccw else rs_recv_cw
                with jax.named_scope(f"pacc#h={h}#d={int(ccw)}"):
                    compute_pacc(ccw, h)
                for q in range(2):
                    qs = pl.ds(q * HH, HH)
                    if h == 0:
                        with jax.named_scope(f"chainstart#q={q}#d={int(ccw)}"):
                            rs_start[:, qs] = pacc[:, qs].astype(jnp.bfloat16)
                            start_rs(0, q, ccw, rs_start.at[:, qs],
                                     rs_recv.at[0, :, qs])
                    else:
                        with jax.named_scope(f"rswait#h={h}#q={q}#d={int(ccw)}"):
                            rdmas[("rs", h - 1, q, ccw)].wait_recv()
                        with jax.named_scope(f"fold#h={h}#q={q}#d={int(ccw)}"):
                            rs_recv[h - 1, :, qs] = (
                                rs_recv[h - 1, :, qs].astype(jnp.float32)
                                + pacc[:, qs]
                            ).astype(jnp.bfloat16)
                            start_rs(h, q, ccw, rs_recv.at[h - 1, :, qs],
                                     rs_recv.at[h, :, qs])

        for q in range(2):
            qs = pl.ds(q * HH, HH)
            with jax.named_scope(f"finwait#q={q}#d=0"):
                rdmas[("rs", 2, q, False)].wait_recv()
            with jax.named_scope(f"finadd#q={q}#d=0"):
                out_ref[:HALF, qs] = (
                    out_ref[:HALF, qs] + rs_recv_cw[2, :, qs].astype(jnp.float32)
                )
            with jax.named_scope(f"finwait#q={q}#d=1"):
                rdmas[("rs", 2, q, True)].wait_recv()
            with jax.named_scope(f"finadd#q={q}#d=1"):
                out_ref[HALF:, qs] = (
                    out_ref[HALF:, qs] + rs_recv_ccw[2, :, qs].astype(jnp.float32)
                )

        with jax.named_scope("drain"):
            for rd in rdmas.values():
                rd.wait_send()

    return pl.pallas_call(
        body,
        out_shape=jax.ShapeDtypeStruct((N_TOK, H), jnp.float32),
        in_specs=[pl.BlockSpec(memory_space=pltpu.VMEM)] * 6,
        out_specs=pl.BlockSpec(memory_space=pltpu.VMEM),
        scratch_shapes=[
            pltpu.VMEM((N_DEV - 1, HALF, D), jnp.bfloat16),
            pltpu.VMEM((N_DEV - 1, HALF, D), jnp.bfloat16),
            pltpu.VMEM((N_DEV - 1, HALF, E), jnp.bfloat16),
            pltpu.VMEM((N_DEV - 1, HALF, E), jnp.bfloat16),
            pltpu.VMEM((HALF, E), jnp.bfloat16),
            pltpu.VMEM((HALF, E), jnp.bfloat16),
            pltpu.VMEM((HALF, H), jnp.bfloat16),
            pltpu.VMEM((HALF, H), jnp.bfloat16),
            pltpu.VMEM((N_DEV - 1, HALF, H), jnp.bfloat16),
            pltpu.VMEM((N_DEV - 1, HALF, H), jnp.bfloat16),
            pltpu.VMEM((HALF, H), jnp.float32),
            pltpu.VMEM((HALF, H), jnp.float32),
            pltpu.SemaphoreType.DMA((3, N_DEV - 1)),
            pltpu.SemaphoreType.DMA((3, N_DEV - 1)),
            pltpu.SemaphoreType.DMA((3, N_DEV - 1)),
            pltpu.SemaphoreType.DMA((3, N_DEV - 1)),
            pltpu.SemaphoreType.DMA((N_DEV - 1, 2)),
            pltpu.SemaphoreType.DMA((N_DEV - 1, 2)),
            pltpu.SemaphoreType.DMA((N_DEV - 1, 2)),
            pltpu.SemaphoreType.DMA((N_DEV - 1, 2)),
            pltpu.VMEM((HALF, E_LOC * D), jnp.bfloat16),
            pltpu.VMEM((HALF, E_LOC * D), jnp.bfloat16),
            pltpu.VMEM((N_TOK, E_LOC * D), jnp.bfloat16),
        ],
        compiler_params=pltpu.CompilerParams(collective_id=0),
    )(x, router_W, route_idx, ew2, xa, xb)


# device time: 76376 ns/iter; 1.0031x vs baseline; 1.0031x over previous
import jax
import jax.numpy as jnp
from jax import lax
from jax.experimental import pallas as pl
from jax.experimental.pallas import tpu as pltpu

N_DEV = 4
E_LOC = 4
E = N_DEV * E_LOC
N_TOK = 1024
HALF = N_TOK // 2
D = 512
H = 1024

KX, KG, KRS = 0, 1, 2


def kernel(x, router_W, route_idx, expert_W):
    ew2 = expert_W.astype(jnp.bfloat16).reshape(E_LOC * D, H)
    x_bf16 = x.astype(jnp.bfloat16)
    xa = x_bf16[:HALF]
    xb = x_bf16[HALF:]

    def body(x_ref, rw_ref, idx_ref, ew_ref, xa_ref, xb_ref, out_ref,
             agx_cw, agx_ccw, agg_cw, agg_ccw, gsrc_cw, gsrc_ccw,
             rs_start_cw, rs_start_ccw, rs_recv_cw, rs_recv_ccw,
             pacc_cw, pacc_ccw, send_cw, recv_cw, send_ccw, recv_ccw,
             rssend_cw, rsrecv_cw, rssend_ccw, rsrecv_ccw,
             xgc_cw, xgc_ccw, xgl):
        my = lax.axis_index("i")
        left = lax.rem(my - 1 + N_DEV, N_DEV)
        right = lax.rem(my + 1, N_DEV)

        barrier_sem = pltpu.get_barrier_semaphore()
        for nbr in (left, right):
            pl.semaphore_signal(
                barrier_sem, inc=1,
                device_id=(nbr,), device_id_type=pl.DeviceIdType.MESH,
            )
        pl.semaphore_wait(barrier_sem, 2)

        def make_copy(kind, h, ccw, src, dst):
            if ccw:
                tgt = left if h < 2 else right
            else:
                tgt = right if h < 2 else left
            return pltpu.make_async_remote_copy(
                src_ref=src,
                dst_ref=dst,
                send_sem=(send_ccw if ccw else send_cw).at[kind, h],
                recv_sem=(recv_ccw if ccw else recv_cw).at[kind, h],
                device_id=(tgt,),
                device_id_type=pl.DeviceIdType.MESH,
            )

        rdmas = {}

        def start(kind, h, ccw, src, dst):
            rd = make_copy(kind, h, ccw, src, dst)
            rdmas[(kind, h, ccw)] = rd
            rd.start()

        HH = H // 2

        def start_rs(h, q, ccw, src, dst):
            rd = pltpu.make_async_remote_copy(
                src_ref=src,
                dst_ref=dst,
                send_sem=(rssend_ccw if ccw else rssend_cw).at[h, q],
                recv_sem=(rsrecv_ccw if ccw else rsrecv_cw).at[h, q],
                device_id=(left if ccw else right,),
                device_id_type=pl.DeviceIdType.MESH,
            )
            rdmas[("rs", h, q, ccw)] = rd
            rd.start()

        start(KX, 0, False, xa_ref, agx_cw.at[0])
        start(KX, 0, True, xb_ref, agx_ccw.at[0])

        sc_gates = jax.named_scope("gates")
        sc_gates.__enter__()
        xf = x_ref[:, :]
        scores = jnp.dot(xf, rw_ref[:, :], preferred_element_type=jnp.float32)
        m = jnp.max(scores, axis=1, keepdims=True)
        p = jnp.exp(scores - m)
        p = p / jnp.sum(p, axis=1, keepdims=True)
        e0 = idx_ref[:, 0:1]
        e1 = idx_ref[:, 1:2]
        lanes = lax.broadcasted_iota(jnp.int32, (N_TOK, E), 1)
        m0 = lanes == e0
        m1 = lanes == e1
        g0 = jnp.sum(jnp.where(m0, p, 0.0), axis=1, keepdims=True)
        g1 = jnp.sum(jnp.where(m1, p, 0.0), axis=1, keepdims=True)
        gs = g0 + g1
        gate = jnp.where(m0, g0 / gs, 0.0) + jnp.where(m1, g1 / gs, 0.0)

        gsrc_cw[:, :] = gate[:HALF].astype(jnp.bfloat16)
        gsrc_ccw[:, :] = gate[HALF:].astype(jnp.bfloat16)
        start(KG, 0, False, gsrc_cw, agg_cw.at[0])
        start(KG, 0, True, gsrc_ccw, agg_ccw.at[0])
        sc_gates.__exit__(None, None, None)

        def gate_block(gr):
            rows = lax.broadcasted_iota(jnp.int32, (E, E_LOC), 0)
            cols = lax.broadcasted_iota(jnp.int32, (E, E_LOC), 1)
            sel = (rows == my * E_LOC + cols).astype(jnp.float32)
            return jnp.dot(gr, sel, preferred_element_type=jnp.float32)

        def compute_pacc(ccw, h):
            agx = agx_ccw if ccw else agx_cw
            agg = agg_ccw if ccw else agg_cw
            pacc = pacc_ccw if ccw else pacc_cw
            xgc = xgc_ccw if ccw else xgc_cw
            xr = agx[h]
            gblk = gate_block(agg[h].astype(jnp.float32))
            for j in range(E_LOC):
                xgc[:, j * D:(j + 1) * D] = (
                    xr * gblk[:, j:j + 1]
                ).astype(jnp.bfloat16)
            pacc[:, :] = jnp.dot(xgc[:, :], ew_ref[:, :],
                                 preferred_element_type=jnp.float32)

        with jax.named_scope("local_block"):
            gblk = gate_block(gate)
            for j in range(E_LOC):
                xgl[:, j * D:(j + 1) * D] = (
                    xf * gblk[:, j:j + 1]
                ).astype(jnp.bfloat16)
            out_ref[:, :] = jnp.dot(xgl[:, :], ew_ref[:, :],
                                    preferred_element_type=jnp.float32)

        for h in range(N_DEV - 1):
            for ccw in (False, True):
                agx = agx_ccw if ccw else agx_cw
                agg = agg_ccw if ccw else agg_cw
                with jax.named_scope(f"agwait#h={h}#d={int(ccw)}"):
                    rdmas[(KX, h, ccw)].wait_recv()
                    rdmas[(KG, h, ccw)].wait_recv()
                if h == 0:
                    start(KX, 1, ccw, agx.at[0], agx.at[1])
                    start(KG, 1, ccw, agg.at[0], agg.at[1])
            for ccw in (False, True):
                pacc = pacc_ccw if ccw else pacc_cw
                rs_start = rs_start_ccw if ccw else rs_start_cw
                rs_recv = rs_recv_ccw if ccw else rs_recv_cw
                with jax.named_scope(f"pacc#h={h}#d={int(ccw)}"):
                    compute_pacc(ccw, h)
                for q in range(2):
                    qs = pl.ds(q * HH, HH)
                    if h == 0:
                        with jax.named_scope(f"chainstart#q={q}#d={int(ccw)}"):
                            rs_start[:, qs] = pacc[:, qs].astype(jnp.bfloat16)
                            start_rs(0, q, ccw, rs_start.at[:, qs],
                                     rs_recv.at[0, :, qs])
                    else:
                        with jax.named_scope(f"rswait#h={h}#q={q}#d={int(ccw)}"):
                            rdmas[("rs", h - 1, q, ccw)].wait_recv()
                        with jax.named_scope(f"fold#h={h}#q={q}#d={int(ccw)}"):
                            rs_recv[h - 1, :, qs] = (
                                rs_recv[h - 1, :, qs].astype(jnp.float32)
                                + pacc[:, qs]
                            ).astype(jnp.bfloat16)
                            start_rs(h, q, ccw, rs_recv.at[h - 1, :, qs],
                                     rs_recv.at[h, :, qs])
            if h == 0:
                start(KX, 2, False, xa_ref, agx_cw.at[2])
                start(KX, 2, True, xb_ref, agx_ccw.at[2])
                start(KG, 2, False, gsrc_cw, agg_cw.at[2])
                start(KG, 2, True, gsrc_ccw, agg_ccw.at[2])

        for q in range(2):
            qs = pl.ds(q * HH, HH)
            with jax.named_scope(f"finwait#q={q}#d=0"):
                rdmas[("rs", 2, q, False)].wait_recv()
            with jax.named_scope(f"finadd#q={q}#d=0"):
                out_ref[:HALF, qs] = (
                    out_ref[:HALF, qs] + rs_recv_cw[2, :, qs].astype(jnp.float32)
                )
            with jax.named_scope(f"finwait#q={q}#d=1"):
                rdmas[("rs", 2, q, True)].wait_recv()
            with jax.named_scope(f"finadd#q={q}#d=1"):
                out_ref[HALF:, qs] = (
                    out_ref[HALF:, qs] + rs_recv_ccw[2, :, qs].astype(jnp.float32)
                )

        with jax.named_scope("drain"):
            for rd in rdmas.values():
                rd.wait_send()

    return pl.pallas_call(
        body,
        out_shape=jax.ShapeDtypeStruct((N_TOK, H), jnp.float32),
        in_specs=[pl.BlockSpec(memory_space=pltpu.VMEM)] * 6,
        out_specs=pl.BlockSpec(memory_space=pltpu.VMEM),
        scratch_shapes=[
            pltpu.VMEM((N_DEV - 1, HALF, D), jnp.bfloat16),
            pltpu.VMEM((N_DEV - 1, HALF, D), jnp.bfloat16),
            pltpu.VMEM((N_DEV - 1, HALF, E), jnp.bfloat16),
            pltpu.VMEM((N_DEV - 1, HALF, E), jnp.bfloat16),
            pltpu.VMEM((HALF, E), jnp.bfloat16),
            pltpu.VMEM((HALF, E), jnp.bfloat16),
            pltpu.VMEM((HALF, H), jnp.bfloat16),
            pltpu.VMEM((HALF, H), jnp.bfloat16),
            pltpu.VMEM((N_DEV - 1, HALF, H), jnp.bfloat16),
            pltpu.VMEM((N_DEV - 1, HALF, H), jnp.bfloat16),
            pltpu.VMEM((HALF, H), jnp.float32),
            pltpu.VMEM((HALF, H), jnp.float32),
            pltpu.SemaphoreType.DMA((3, N_DEV - 1)),
            pltpu.SemaphoreType.DMA((3, N_DEV - 1)),
            pltpu.SemaphoreType.DMA((3, N_DEV - 1)),
            pltpu.SemaphoreType.DMA((3, N_DEV - 1)),
            pltpu.SemaphoreType.DMA((N_DEV - 1, 2)),
            pltpu.SemaphoreType.DMA((N_DEV - 1, 2)),
            pltpu.SemaphoreType.DMA((N_DEV - 1, 2)),
            pltpu.SemaphoreType.DMA((N_DEV - 1, 2)),
            pltpu.VMEM((HALF, E_LOC * D), jnp.bfloat16),
            pltpu.VMEM((HALF, E_LOC * D), jnp.bfloat16),
            pltpu.VMEM((N_TOK, E_LOC * D), jnp.bfloat16),
        ],
        compiler_params=pltpu.CompilerParams(collective_id=0),
    )(x, router_W, route_idx, ew2, xa, xb)


# device time: 74540 ns/iter; 1.0278x vs baseline; 1.0246x over previous
import jax
import jax.numpy as jnp
from jax import lax
from jax.experimental import pallas as pl
from jax.experimental.pallas import tpu as pltpu

N_DEV = 4
E_LOC = 4
E = N_DEV * E_LOC
N_TOK = 1024
HALF = N_TOK // 2
D = 512
H = 1024

KX, KG, KRS = 0, 1, 2


def kernel(x, router_W, route_idx, expert_W):
    ew2 = expert_W.astype(jnp.bfloat16).reshape(E_LOC * D, H)
    x_bf16 = x.astype(jnp.bfloat16)
    xa = x_bf16[:HALF]
    xb = x_bf16[HALF:]

    def body(x_ref, rw_ref, idx_ref, ew_ref, xa_ref, xb_ref, out_ref,
             agx_cw, agx_ccw, agg_cw, agg_ccw, gsrc_cw, gsrc_ccw,
             rs_start_cw, rs_start_ccw, rs_recv_cw, rs_recv_ccw,
             pacc_cw, pacc_ccw, send_cw, recv_cw, send_ccw, recv_ccw,
             rssend_cw, rsrecv_cw, rssend_ccw, rsrecv_ccw,
             xgc_cw, xgc_ccw, xgl):
        my = lax.axis_index("i")
        left = lax.rem(my - 1 + N_DEV, N_DEV)
        right = lax.rem(my + 1, N_DEV)

        barrier_sem = pltpu.get_barrier_semaphore()
        for nbr in (left, right):
            pl.semaphore_signal(
                barrier_sem, inc=1,
                device_id=(nbr,), device_id_type=pl.DeviceIdType.MESH,
            )
        pl.semaphore_wait(barrier_sem, 2)

        def make_copy(kind, h, ccw, src, dst):
            if ccw:
                tgt = left if h < 2 else right
            else:
                tgt = right if h < 2 else left
            return pltpu.make_async_remote_copy(
                src_ref=src,
                dst_ref=dst,
                send_sem=(send_ccw if ccw else send_cw).at[kind, h],
                recv_sem=(recv_ccw if ccw else recv_cw).at[kind, h],
                device_id=(tgt,),
                device_id_type=pl.DeviceIdType.MESH,
            )

        rdmas = {}

        def start(kind, h, ccw, src, dst):
            rd = make_copy(kind, h, ccw, src, dst)
            rdmas[(kind, h, ccw)] = rd
            rd.start()

        HH = H // 2

        def start_rs(h, q, ccw, src, dst):
            rd = pltpu.make_async_remote_copy(
                src_ref=src,
                dst_ref=dst,
                send_sem=(rssend_ccw if ccw else rssend_cw).at[h, q],
                recv_sem=(rsrecv_ccw if ccw else rsrecv_cw).at[h, q],
                device_id=(left if ccw else right,),
                device_id_type=pl.DeviceIdType.MESH,
            )
            rdmas[("rs", h, q, ccw)] = rd
            rd.start()

        start(KX, 0, False, xa_ref, agx_cw.at[0])
        start(KX, 0, True, xb_ref, agx_ccw.at[0])

        sc_gates = jax.named_scope("gates")
        sc_gates.__enter__()
        xf = x_ref[:, :]
        scores = jnp.dot(xf, rw_ref[:, :], preferred_element_type=jnp.float32)
        m = jnp.max(scores, axis=1, keepdims=True)
        p = jnp.exp(scores - m)
        p = p / jnp.sum(p, axis=1, keepdims=True)
        e0 = idx_ref[:, 0:1]
        e1 = idx_ref[:, 1:2]
        lanes = lax.broadcasted_iota(jnp.int32, (N_TOK, E), 1)
        m0 = lanes == e0
        m1 = lanes == e1
        g0 = jnp.sum(jnp.where(m0, p, 0.0), axis=1, keepdims=True)
        g1 = jnp.sum(jnp.where(m1, p, 0.0), axis=1, keepdims=True)
        gs = g0 + g1
        gate = jnp.where(m0, g0 / gs, 0.0) + jnp.where(m1, g1 / gs, 0.0)

        gsrc_cw[:, :] = gate[:HALF].astype(jnp.bfloat16)
        gsrc_ccw[:, :] = gate[HALF:].astype(jnp.bfloat16)
        start(KG, 0, False, gsrc_cw, agg_cw.at[0])
        start(KG, 0, True, gsrc_ccw, agg_ccw.at[0])
        sc_gates.__exit__(None, None, None)

        def gate_block(gr):
            rows = lax.broadcasted_iota(jnp.int32, (E, E_LOC), 0)
            cols = lax.broadcasted_iota(jnp.int32, (E, E_LOC), 1)
            sel = (rows == my * E_LOC + cols).astype(jnp.float32)
            return jnp.dot(gr, sel, preferred_element_type=jnp.float32)

        def compute_pacc(ccw, h):
            agx = agx_ccw if ccw else agx_cw
            agg = agg_ccw if ccw else agg_cw
            pacc = pacc_ccw if ccw else pacc_cw
            xgc = xgc_ccw if ccw else xgc_cw
            xr = agx[h]
            gblk = gate_block(agg[h].astype(jnp.float32))
            for j in range(E_LOC):
                xgc[:, j * D:(j + 1) * D] = (
                    xr * gblk[:, j:j + 1]
                ).astype(jnp.bfloat16)
            pacc[:, :] = jnp.dot(xgc[:, :], ew_ref[:, :],
                                 preferred_element_type=jnp.float32)

        with jax.named_scope("local_block"):
            gblk = gate_block(gate)
            for j in range(E_LOC):
                xgl[:, j * D:(j + 1) * D] = (
                    xf * gblk[:, j:j + 1]
                ).astype(jnp.bfloat16)
            out_ref[:, :] = jnp.dot(xgl[:, :], ew_ref[:, :],
                                    preferred_element_type=jnp.float32)

        start(KX, 2, False, xa_ref, agx_cw.at[2])
        start(KX, 2, True, xb_ref, agx_ccw.at[2])
        start(KG, 2, False, gsrc_cw, agg_cw.at[2])
        start(KG, 2, True, gsrc_ccw, agg_ccw.at[2])

        for h in range(N_DEV - 1):
            for ccw in (False, True):
                agx = agx_ccw if ccw else agx_cw
                agg = agg_ccw if ccw else agg_cw
                with jax.named_scope(f"agwait#h={h}#d={int(ccw)}"):
                    rdmas[(KX, h, ccw)].wait_recv()
                    rdmas[(KG, h, ccw)].wait_recv()
                if h == 0:
                    start(KX, 1, ccw, agx.at[0], agx.at[1])
                    start(KG, 1, ccw, agg.at[0], agg.at[1])
            for ccw in (False, True):
                pacc = pacc_ccw if ccw else pacc_cw
                rs_start = rs_start_ccw if ccw else rs_start_cw
                rs_recv = rs_recv_ccw if ccw else rs_recv_cw
                with jax.named_scope(f"pacc#h={h}#d={int(ccw)}"):
                    compute_pacc(ccw, h)
                for q in range(2):
                    qs = pl.ds(q * HH, HH)
                    if h == 0:
                        with jax.named_scope(f"chainstart#q={q}#d={int(ccw)}"):
                            rs_start[:, qs] = pacc[:, qs].astype(jnp.bfloat16)
                            start_rs(0, q, ccw, rs_start.at[:, qs],
                                     rs_recv.at[0, :, qs])
                    else:
                        with jax.named_scope(f"rswait#h={h}#q={q}#d={int(ccw)}"):
                            rdmas[("rs", h - 1, q, ccw)].wait_recv()
                        with jax.named_scope(f"fold#h={h}#q={q}#d={int(ccw)}"):
                            rs_recv[h - 1, :, qs] = (
                                rs_recv[h - 1, :, qs].astype(jnp.float32)
                                + pacc[:, qs]
                            ).astype(jnp.bfloat16)
                            start_rs(h, q, ccw, rs_recv.at[h - 1, :, qs],
                                     rs_recv.at[h, :, qs])
        for q in range(2):
            qs = pl.ds(q * HH, HH)
            with jax.named_scope(f"finwait#q={q}#d=0"):
                rdmas[("rs", 2, q, False)].wait_recv()
            with jax.named_scope(f"finadd#q={q}#d=0"):
                out_ref[:HALF, qs] = (
                    out_ref[:HALF, qs] + rs_recv_cw[2, :, qs].astype(jnp.float32)
                )
            with jax.named_scope(f"finwait#q={q}#d=1"):
                rdmas[("rs", 2, q, True)].wait_recv()
            with jax.named_scope(f"finadd#q={q}#d=1"):
                out_ref[HALF:, qs] = (
                    out_ref[HALF:, qs] + rs_recv_ccw[2, :, qs].astype(jnp.float32)
                )

        with jax.named_scope("drain"):
            for rd in rdmas.values():
                rd.wait_send()

    return pl.pallas_call(
        body,
        out_shape=jax.ShapeDtypeStruct((N_TOK, H), jnp.float32),
        in_specs=[pl.BlockSpec(memory_space=pltpu.VMEM)] * 6,
        out_specs=pl.BlockSpec(memory_space=pltpu.VMEM),
        scratch_shapes=[
            pltpu.VMEM((N_DEV - 1, HALF, D), jnp.bfloat16),
            pltpu.VMEM((N_DEV - 1, HALF, D), jnp.bfloat16),
            pltpu.VMEM((N_DEV - 1, HALF, E), jnp.bfloat16),
            pltpu.VMEM((N_DEV - 1, HALF, E), jnp.bfloat16),
            pltpu.VMEM((HALF, E), jnp.bfloat16),
            pltpu.VMEM((HALF, E), jnp.bfloat16),
            pltpu.VMEM((HALF, H), jnp.bfloat16),
            pltpu.VMEM((HALF, H), jnp.bfloat16),
            pltpu.VMEM((N_DEV - 1, HALF, H), jnp.bfloat16),
            pltpu.VMEM((N_DEV - 1, HALF, H), jnp.bfloat16),
            pltpu.VMEM((HALF, H), jnp.float32),
            pltpu.VMEM((HALF, H), jnp.float32),
            pltpu.SemaphoreType.DMA((3, N_DEV - 1)),
            pltpu.SemaphoreType.DMA((3, N_DEV - 1)),
            pltpu.SemaphoreType.DMA((3, N_DEV - 1)),
            pltpu.SemaphoreType.DMA((3, N_DEV - 1)),
            pltpu.SemaphoreType.DMA((N_DEV - 1, 2)),
            pltpu.SemaphoreType.DMA((N_DEV - 1, 2)),
            pltpu.SemaphoreType.DMA((N_DEV - 1, 2)),
            pltpu.SemaphoreType.DMA((N_DEV - 1, 2)),
            pltpu.VMEM((HALF, E_LOC * D), jnp.bfloat16),
            pltpu.VMEM((HALF, E_LOC * D), jnp.bfloat16),
            pltpu.VMEM((N_TOK, E_LOC * D), jnp.bfloat16),
        ],
        compiler_params=pltpu.CompilerParams(collective_id=0),
    )(x, router_W, route_idx, ew2, xa, xb)
